# baseline (speedup 1.0000x reference)
import jax
import jax.numpy as jnp
from jax import lax
from jax.experimental import pallas as pl
from jax.experimental.pallas import tpu as pltpu
from jax.experimental.pallas import tpu_sc as plsc

N = 10000
E = 320000
D = 128
OUT = 128

NC = 2
NS = 16
NW = NC * NS

NBUF = 4
CHUNK = 80
EPW = E // NW
HALF = 64 * CHUNK
NCHUNK = EPW // CHUNK
NPAD = 10240
ROWS_PT = NPAD // NS


def _scale_rows(buf, wc):
    def scale_group(g, _):
        w16 = wc[pl.ds(g * 16, 16)]
        for l in range(16):
            wv = lax.gather(
                w16, jnp.full((16, 1), l, jnp.int32),
                lax.GatherDimensionNumbers(offset_dims=(),
                                           collapsed_slice_dims=(0,),
                                           start_index_map=(0,)),
                (1,), mode=lax.GatherScatterMode.PROMISE_IN_BOUNDS)
            r = g * 16 + l
            for u in range(D // 16):
                buf[r, pl.ds(u * 16, 16)] = buf[r, pl.ds(u * 16, 16)] * wv
        return 0
    lax.fori_loop(0, CHUNK // 16, scale_group, 0)


def _sc_agg_body(h_hbm, src_hbm, dst_hbm, w_hbm, out_hbm,
                 src_v, dst_c, wc_v, buf_v, agg_sh,
                 sems, wsems, dsems, ssems, zsem):
    cid = lax.axis_index("c")
    sid = lax.axis_index("s")
    wid = cid * NS + sid

    def _src_slice(j):
        return src_v.at[pl.ds(lax.rem(j, 64) * CHUNK, CHUNK)]

    def start_fetch(j, b):
        pltpu.async_copy(w_hbm.at[wid, j], wc_v.at[b], wsems.at[b])
        pltpu.async_copy(dst_hbm.at[wid, j], dst_c.at[b], dsems.at[b])
        pltpu.async_copy(h_hbm.at[_src_slice(j)], buf_v.at[b], sems.at[b])

    def wait_fetch(j, b):
        pltpu.make_async_copy(w_hbm.at[wid, j], wc_v.at[b], wsems.at[b]).wait()
        pltpu.make_async_copy(dst_hbm.at[wid, j], dst_c.at[b],
                              dsems.at[b]).wait()
        pltpu.make_async_copy(h_hbm.at[_src_slice(j)], buf_v.at[b],
                              sems.at[b]).wait()

    def start_scatter(b):
        pltpu.async_copy(buf_v.at[b], agg_sh.at[dst_c.at[b, 0]], ssems.at[b],
                         add=True)

    def wait_scatter(b):
        pltpu.make_async_copy(buf_v.at[b], agg_sh.at[dst_c.at[b, 0]],
                              ssems.at[b]).wait()

    pltpu.sync_copy(src_hbm.at[wid, pl.ds(0, HALF)], src_v)
    start_fetch(0, 0)
    start_fetch(1, 1)

    def zero_row(r, _):
        z = jnp.zeros((16,), jnp.float32)
        for u in range(D // 16):
            buf_v[3, r, pl.ds(u * 16, 16)] = z
        return 0
    lax.fori_loop(0, CHUNK, zero_row, 0)
    for t in range(ROWS_PT // CHUNK):
        pltpu.async_copy(buf_v.at[3],
                         agg_sh.at[pl.ds(sid * ROWS_PT + t * CHUNK, CHUNK)],
                         zsem)
    for t in range(ROWS_PT // CHUNK):
        pltpu.make_async_copy(buf_v.at[3],
                              agg_sh.at[pl.ds(sid * ROWS_PT + t * CHUNK,
                                              CHUNK)],
                              zsem).wait()
    plsc.subcore_barrier()

    def step(j, b, fetch_ahead):
        wait_fetch(j, b)
        _scale_rows(buf_v.at[b], wc_v.at[b, 0])
        wait_scatter((b + 2) % NBUF)
        if fetch_ahead:
            start_fetch(j + 2, (b + 2) % NBUF)
        start_scatter(b)

    def do_quad(jt, _):
        for b in range(NBUF):
            j = jt * NBUF + b
            wait_fetch(j, b)
            _scale_rows(buf_v.at[b], wc_v.at[b, 0])

            @pl.when(j >= 2)
            def _():
                wait_scatter((b + 2) % NBUF)
            start_fetch(j + 2, (b + 2) % NBUF)

            @pl.when(j == 61)
            def _():
                pltpu.sync_copy(src_hbm.at[wid, pl.ds(HALF, HALF)],
                                src_v.at[pl.ds(0, HALF)])
            start_scatter(b)
        return 0
    lax.fori_loop(0, 120 // NBUF, do_quad, 0)
    step(120, 0, True)
    step(121, 1, True)
    step(122, 2, True)
    step(123, 3, False)
    step(124, 0, False)
    wait_scatter(3)
    wait_scatter(0)

    plsc.subcore_barrier()
    pltpu.sync_copy(agg_sh.at[pl.ds(sid * ROWS_PT, ROWS_PT)],
                    out_hbm.at[cid, pl.ds(sid * ROWS_PT, ROWS_PT)])


@jax.jit
def _sc_aggregate(h, src, dst, w):
    mesh = plsc.VectorSubcoreMesh(core_axis_name="c", subcore_axis_name="s")
    return pl.kernel(
        _sc_agg_body,
        out_type=jax.ShapeDtypeStruct((NC, NPAD, D), jnp.float32),
        mesh=mesh,
        scratch_types=[
            pltpu.VMEM((HALF,), jnp.int32),
            pltpu.VMEM((NBUF, 1, CHUNK), jnp.int32),
            pltpu.VMEM((NBUF, 1, CHUNK), jnp.float32),
            pltpu.VMEM((NBUF, CHUNK, D), jnp.float32),
            pltpu.VMEM_SHARED((NPAD, D), jnp.float32),
            pltpu.SemaphoreType.DMA((NBUF,)),
            pltpu.SemaphoreType.DMA((NBUF,)),
            pltpu.SemaphoreType.DMA((NBUF,)),
            pltpu.SemaphoreType.DMA((NBUF,)),
            pltpu.SemaphoreType.DMA,
        ],
    )(h, src, dst, w)


ROWB = 400


def _linear_body(h_b, p_b, w1t_b, w2t_b, b_b, out_b):
    agg = p_b[0] + p_b[1]
    acc = jnp.dot(h_b[...], w1t_b[...], preferred_element_type=jnp.float32)
    acc += jnp.dot(agg, w2t_b[...], preferred_element_type=jnp.float32)
    out_b[...] = acc + b_b[...]


@jax.jit
def _linear(h, partials, w1t, w2t, b2d):
    grid = (N // ROWB,)
    return pl.pallas_call(
        _linear_body,
        grid=grid,
        in_specs=[
            pl.BlockSpec((ROWB, D), lambda i: (i, 0)),
            pl.BlockSpec((NC, ROWB, D), lambda i: (0, i, 0)),
            pl.BlockSpec((D, OUT), lambda i: (0, 0)),
            pl.BlockSpec((D, OUT), lambda i: (0, 0)),
            pl.BlockSpec((1, OUT), lambda i: (0, 0)),
        ],
        out_specs=pl.BlockSpec((ROWB, OUT), lambda i: (i, 0)),
        out_shape=jax.ShapeDtypeStruct((N, OUT), jnp.float32),
    )(h, partials, w1t, w2t, b2d)


def kernel(h, edge_index, edge_w, W, b):
    src = edge_index[0].astype(jnp.int32).reshape(NW, EPW)
    src = jnp.pad(src, ((0, 0), (0, 2 * HALF - EPW)))
    dst = edge_index[1].astype(jnp.int32).reshape(NW, NCHUNK, 1, CHUNK)
    w = edge_w.reshape(NW, NCHUNK, 1, CHUNK).astype(jnp.float32)
    partials = _sc_aggregate(h, src, dst, w)
    w1t = W[:, :D].T
    w2t = W[:, D:].T
    return _linear(h, partials, w1t, w2t, b.reshape(1, OUT))
